# baseline (speedup 1.0000x reference)
import functools

import jax
import jax.numpy as jnp
from jax import lax
from jax.experimental import pallas as pl
from jax.experimental.pallas import tpu as pltpu
from jax.experimental.pallas import tpu_sc as plsc

_K = 51
_LN2 = 0.6931471805599453


def _inv_sqrt(x):
    bits = lax.bitcast_convert_type(x, jnp.int32)
    y = lax.bitcast_convert_type(
        jnp.int32(0x5F3759DF) - (bits >> 1), jnp.float32)
    for _ in range(3):
        y = y * (1.5 - 0.5 * x * y * y)
    return y


def _log(x):
    bits = lax.bitcast_convert_type(x, jnp.int32)
    e = (bits >> 23) - 127
    m = lax.bitcast_convert_type(
        (bits & jnp.int32(0x007FFFFF)) | jnp.int32(0x3F800000), jnp.float32)
    t = (m - 1.0) / (m + 1.0)
    t2 = t * t
    p = 1.0 + t2 * (1.0 / 3.0 + t2 * (1.0 / 5.0 + t2 * (1.0 / 7.0 + t2 / 9.0)))
    return e.astype(jnp.float32) * _LN2 + 2.0 * t * p


def _make_sc_kernel(B, V, D, NN):
    try:
        info = plsc.get_sparse_core_info()
        NC, NS, L = info.num_cores, info.num_subcores, info.num_lanes
    except ValueError:
        NC, NS, L = 2, 16, 16
    NW = NC * NS
    bpw = B // NW

    mesh = plsc.VectorSubcoreMesh(
        core_axis_name="c", subcore_axis_name="s",
        num_cores=NC, num_subcores=NS)

    @functools.partial(
        pl.kernel,
        out_type=[
            jax.ShapeDtypeStruct((B,), jnp.float32),
            jax.ShapeDtypeStruct((B,), jnp.float32),
        ],
        mesh=mesh,
        scratch_types=[
            pltpu.VMEM((bpw,), jnp.int32),
            pltpu.VMEM((bpw, D), jnp.float32),
            pltpu.VMEM((L, L), jnp.int32),
            pltpu.VMEM((4 * L, D), jnp.float32),
            pltpu.SemaphoreType.DMA,
            pltpu.SemaphoreType.DMA,
            pltpu.VMEM((bpw,), jnp.float32),
            pltpu.VMEM((bpw,), jnp.float32),
        ],
        compiler_params=pltpu.CompilerParams(
            needs_layout_passes=False, use_tc_tiling_on_sc=False),
    )
    def sc_kernel(labels_hbm, recv_hbm, nnsc_hbm, emb_hbm, loss_hbm, acc_hbm,
                  lab_v, recv_v, nnw_v, rows_v, sem1, sem2, loss_v, acc_v):
        wid = lax.axis_index("s") * NC + lax.axis_index("c")
        base = wid * bpw
        lane = lax.iota(jnp.int32, L)
        kmask3 = lane < (_K - 3 * L)

        pltpu.sync_copy(labels_hbm.at[pl.ds(base, bpw)], lab_v)
        pltpu.sync_copy(recv_hbm.at[pl.ds(base, bpw)], recv_v)

        def hop1(b):
            labv = plsc.load_gather(lab_v, [jnp.full((L,), b, jnp.int32)])
            wordbase = labv * NN
            cidx = (wordbase >> 4) + jnp.minimum(lane, 4)
            pltpu.async_copy(nnsc_hbm.at[cidx], nnw_v, sem1).wait()
            return wordbase & 15

        def hop2(off):
            for c in range(4):
                wg = off + lane + c * L
                wg = jnp.minimum(wg, off + (_K - 1))
                ids = plsc.load_gather(nnw_v, [wg >> 4, wg & 15])
                pltpu.make_async_copy(
                    emb_hbm.at[ids], rows_v.at[pl.ds(c * L, L)], sem2).start()
            for c in range(4):
                pltpu.make_async_copy(
                    emb_hbm.at[lane], rows_v.at[pl.ds(c * L, L)], sem2).wait()

        def compute(b):
            r = [recv_v[b, pl.ds(c * L, L)] for c in range(D // L)]
            norm2 = jnp.sum(sum(rc * rc for rc in r))
            inv_norm = _inv_sqrt(norm2)

            acc = [jnp.zeros((L,), jnp.float32) for _ in range(4)]
            for d in range(D):
                rd = r[d // L][d % L]
                dvec = jnp.full((L,), d, jnp.int32)
                for kc in range(4):
                    kidx = lane + kc * L
                    mask = kmask3 if kc == 3 else None
                    vals = plsc.load_gather(rows_v, [kidx, dvec], mask=mask)
                    acc[kc] = acc[kc] + vals * rd

            cos = [a * inv_norm for a in acc]
            cos3 = jnp.where(kmask3, cos[3], -1e30)
            m = jnp.max(jnp.maximum(jnp.maximum(cos[0], cos[1]),
                                    jnp.maximum(cos[2], cos3)))
            es = [jnp.exp(c - m) for c in cos[:3]]
            e3 = jnp.where(kmask3, jnp.exp(cos3 - m), 0.0)
            s = jnp.sum(es[0] + es[1] + es[2] + e3)
            logs = _log(jnp.full((L,), s, jnp.float32))
            loss = logs[0] + m - cos[0][0]

            rmax = jnp.max(jnp.maximum(jnp.maximum(r[0], r[1]),
                                       jnp.maximum(r[2], r[3])))
            accv = (r[0][0] >= rmax).astype(jnp.float32)
            return loss, accv

        def body(b, carry):
            lv, av = carry
            off = hop1(b)
            hop2(off)
            loss, accv = compute(b)
            sel = lane == (b & (L - 1))
            lv = jnp.where(sel, loss, lv)
            av = jnp.where(sel, accv, av)

            @pl.when((b & (L - 1)) == (L - 1))
            def _():
                o = pl.multiple_of((b // L) * L, L)
                loss_v[pl.ds(o, L)] = lv
                acc_v[pl.ds(o, L)] = av

            return lv, av

        zero = jnp.zeros((L,), jnp.float32)
        lax.fori_loop(0, bpw, body, (zero, zero))

        pltpu.sync_copy(loss_v, loss_hbm.at[pl.ds(base, bpw)])
        pltpu.sync_copy(acc_v, acc_hbm.at[pl.ds(base, bpw)])

    return sc_kernel


def kernel(_sender_input, _message, _receiver_input, receiver_output,
           _labels, _aux_input, train_emb, train_nns, dev_emb, dev_nns):
    B, D = receiver_output.shape
    V, NN = train_nns.shape
    nns_chunks = train_nns.reshape(V * NN // 16, 16)
    sc = _make_sc_kernel(B, V, D, NN)
    loss, acc = sc(_labels, receiver_output, nns_chunks, train_emb)
    return (loss, acc)


# speedup vs baseline: 1.0658x; 1.0658x over previous
import functools

import jax
import jax.numpy as jnp
from jax import lax
from jax.experimental import pallas as pl
from jax.experimental.pallas import tpu as pltpu
from jax.experimental.pallas import tpu_sc as plsc

_K = 51
_LN2 = 0.6931471805599453


def _inv_sqrt(x):
    bits = lax.bitcast_convert_type(x, jnp.int32)
    y = lax.bitcast_convert_type(
        jnp.int32(0x5F3759DF) - (bits >> 1), jnp.float32)
    for _ in range(3):
        y = y * (1.5 - 0.5 * x * y * y)
    return y


def _log(x):
    bits = lax.bitcast_convert_type(x, jnp.int32)
    e = (bits >> 23) - 127
    m = lax.bitcast_convert_type(
        (bits & jnp.int32(0x007FFFFF)) | jnp.int32(0x3F800000), jnp.float32)
    t = (m - 1.0) / (m + 1.0)
    t2 = t * t
    p = 1.0 + t2 * (1.0 / 3.0 + t2 * (1.0 / 5.0 + t2 * (1.0 / 7.0 + t2 / 9.0)))
    return e.astype(jnp.float32) * _LN2 + 2.0 * t * p


def _make_sc_kernel(B, V, D, NN):
    try:
        info = plsc.get_sparse_core_info()
        NC, NS, L = info.num_cores, info.num_subcores, info.num_lanes
    except ValueError:
        NC, NS, L = 2, 16, 16
    NW = NC * NS
    bpw = B // NW
    NG = bpw // L
    NCH = (_K + 2 * (L - 1)) // L

    mesh = plsc.VectorSubcoreMesh(
        core_axis_name="c", subcore_axis_name="s",
        num_cores=NC, num_subcores=NS)

    @functools.partial(
        pl.kernel,
        out_type=[
            jax.ShapeDtypeStruct((B,), jnp.float32),
            jax.ShapeDtypeStruct((B,), jnp.float32),
        ],
        mesh=mesh,
        scratch_types=[
            pltpu.VMEM((bpw,), jnp.int32),
            pltpu.VMEM((bpw, D), jnp.float32),
            pltpu.VMEM((2, NCH, L, L), jnp.int32),
            pltpu.VMEM((L, 4 * L, D), jnp.float32),
            [pltpu.SemaphoreType.DMA for _ in range(2)],
            [pltpu.SemaphoreType.DMA for _ in range(L)],
            pltpu.VMEM((bpw,), jnp.float32),
            pltpu.VMEM((bpw,), jnp.float32),
        ],
        compiler_params=pltpu.CompilerParams(
            needs_layout_passes=False, use_tc_tiling_on_sc=False),
    )
    def sc_kernel(labels_hbm, recv_hbm, nnsc_hbm, emb_hbm, loss_hbm, acc_hbm,
                  lab_v, recv_v, nnw_v, rows_v, h1sems, rsems,
                  loss_v, acc_v):
        wid = lax.axis_index("s") * NC + lax.axis_index("c")
        base = wid * bpw
        lane = lax.iota(jnp.int32, L)
        kmask3 = lane < (_K - 3 * L)

        pltpu.sync_copy(labels_hbm.at[pl.ds(base, bpw)], lab_v)
        pltpu.sync_copy(recv_hbm.at[pl.ds(base, bpw)], recv_v)

        def hop1_issue(g, slot):
            labv = lab_v[pl.ds(pl.multiple_of(g * L, L), L)]
            cbase = (labv * NN) >> 4
            for c in range(NCH):
                pltpu.make_async_copy(
                    nnsc_hbm.at[cbase + c], nnw_v.at[slot, c],
                    h1sems[slot]).start()

        def hop1_wait(slot):
            for c in range(NCH):
                pltpu.make_async_copy(
                    nnsc_hbm.at[lane], nnw_v.at[slot, c],
                    h1sems[slot]).wait()

        def hop2_issue(gsel, bi, off):
            for kc in range(4):
                wg = jnp.minimum(off + lane + kc * L, off + (_K - 1))
                ids = plsc.load_gather(
                    nnw_v, [jnp.full((L,), gsel, jnp.int32), wg >> 4,
                            jnp.full((L,), bi, jnp.int32), wg & 15])
                pltpu.make_async_copy(
                    emb_hbm.at[ids],
                    rows_v.at[bi, pl.ds(kc * L, L)], rsems[bi]).start()

        def rows_wait(bi):
            for kc in range(4):
                pltpu.make_async_copy(
                    emb_hbm.at[lane],
                    rows_v.at[bi, pl.ds(kc * L, L)], rsems[bi]).wait()

        def compute(b, bi):
            r = [recv_v[b, pl.ds(c * L, L)] for c in range(D // L)]
            norm2 = jnp.sum(sum(rc * rc for rc in r))
            inv_norm = _inv_sqrt(norm2)
            biv = jnp.full((L,), bi, jnp.int32)
            bv = jnp.full((L,), b, jnp.int32)

            def dbody(dj, acc):
                out = list(acc)
                for i in range(8):
                    d = dj * 8 + i
                    rd = plsc.load_gather(
                        recv_v, [bv, jnp.full((L,), d, jnp.int32)])
                    dvec = jnp.full((L,), d, jnp.int32)
                    for kc in range(4):
                        kidx = lane + kc * L
                        mask = kmask3 if kc == 3 else None
                        vals = plsc.load_gather(
                            rows_v, [biv, kidx, dvec], mask=mask)
                        out[kc] = out[kc] + vals * rd
                return tuple(out)

            zero = jnp.zeros((L,), jnp.float32)
            acc = lax.fori_loop(0, D // 8, dbody, (zero, zero, zero, zero))

            cos = [a * inv_norm for a in acc]
            cos3 = jnp.where(kmask3, cos[3], -1e30)
            m = jnp.max(jnp.maximum(jnp.maximum(cos[0], cos[1]),
                                    jnp.maximum(cos[2], cos3)))
            es = [jnp.exp(c - m) for c in cos[:3]]
            e3 = jnp.where(kmask3, jnp.exp(cos3 - m), 0.0)
            s = jnp.sum(es[0] + es[1] + es[2] + e3)
            logs = _log(jnp.full((L,), s, jnp.float32))
            loss = logs[0] + m - cos[0][0]

            rmax = jnp.max(jnp.maximum(jnp.maximum(r[0], r[1]),
                                       jnp.maximum(r[2], r[3])))
            accv = (r[0][0] >= rmax).astype(jnp.float32)
            return loss, accv

        hop1_issue(0, 0)

        def gbody(g, carry):
            even = (g & 1) == 0
            gsel = g & 1

            @pl.when(even)
            def _():
                hop1_wait(0)

            @pl.when(jnp.logical_not(even))
            def _():
                hop1_wait(1)

            @pl.when(jnp.logical_and(g + 1 < NG, even))
            def _():
                hop1_issue(g + 1, 1)

            @pl.when(jnp.logical_and(g + 1 < NG, jnp.logical_not(even)))
            def _():
                hop1_issue(g + 1, 0)

            labv = lab_v[pl.ds(pl.multiple_of(g * L, L), L)]
            off16 = (labv * NN) & 15

            for bi in range(L):
                hop2_issue(gsel, bi, off16[bi])

            lv = jnp.zeros((L,), jnp.float32)
            av = jnp.zeros((L,), jnp.float32)
            for bi in range(L):
                rows_wait(bi)
                loss, accv = compute(g * L + bi, bi)
                sel = lane == bi
                lv = jnp.where(sel, loss, lv)
                av = jnp.where(sel, accv, av)

            o = pl.multiple_of(g * L, L)
            loss_v[pl.ds(o, L)] = lv
            acc_v[pl.ds(o, L)] = av
            return carry

        lax.fori_loop(0, NG, gbody, 0)

        pltpu.sync_copy(loss_v, loss_hbm.at[pl.ds(base, bpw)])
        pltpu.sync_copy(acc_v, acc_hbm.at[pl.ds(base, bpw)])

    return sc_kernel


def kernel(_sender_input, _message, _receiver_input, receiver_output,
           _labels, _aux_input, train_emb, train_nns, dev_emb, dev_nns):
    B, D = receiver_output.shape
    V, NN = train_nns.shape
    nns_chunks = train_nns.reshape(V * NN // 16, 16)
    sc = _make_sc_kernel(B, V, D, NN)
    loss, acc = sc(_labels, receiver_output, nns_chunks, train_emb)
    return (loss, acc)
